# Initial kernel scaffold; baseline (speedup 1.0000x reference)
import jax
import jax.numpy as jnp
from jax.experimental import pallas as pl


def kernel(node_memory, node_features, edge_index, edge_features, time_encoding, W_read, b_read, W_msg, b_msg, W_agg, b_agg, W_upd, b_upd, W_gate, b_gate):
    raise NotImplementedError("write your pallas kernel here")


# speedup vs baseline: 2.4109x; 2.4109x over previous
import functools

import jax
import jax.numpy as jnp
from jax import lax
from jax.experimental import pallas as pl
from jax.experimental.pallas import tpu as pltpu
from jax.experimental.pallas import tpu_sc as plsc

LANES = 16
CB = 128
NW = 32


def _node_prep_body(mem_ref, feat_ref, wr1, wr2, br, wmn, nr_ref, npre_ref):
    nr = jnp.maximum(
        jnp.dot(mem_ref[...], wr1[...], preferred_element_type=jnp.float32)
        + jnp.dot(feat_ref[...], wr2[...], preferred_element_type=jnp.float32)
        + br[...],
        0.0,
    )
    nr_ref[...] = nr
    npre_ref[...] = jnp.dot(nr, wmn[...], preferred_element_type=jnp.float32)


def _node_prep(node_memory, node_features, W_read, b_read, wmn, blk):
    n, mem = node_memory.shape
    feat = node_features.shape[1]
    grid = pl.cdiv(n, blk)
    return pl.pallas_call(
        _node_prep_body,
        grid=(grid,),
        in_specs=[
            pl.BlockSpec((blk, mem), lambda i: (i, 0)),
            pl.BlockSpec((blk, feat), lambda i: (i, 0)),
            pl.BlockSpec((mem, mem), lambda i: (0, 0)),
            pl.BlockSpec((feat, mem), lambda i: (0, 0)),
            pl.BlockSpec((1, mem), lambda i: (0, 0)),
            pl.BlockSpec((mem, mem), lambda i: (0, 0)),
        ],
        out_specs=[
            pl.BlockSpec((blk, mem), lambda i: (i, 0)),
            pl.BlockSpec((blk, mem), lambda i: (i, 0)),
        ],
        out_shape=[
            jax.ShapeDtypeStruct((n, mem), jnp.float32),
            jax.ShapeDtypeStruct((n, mem), jnp.float32),
        ],
    )(node_memory, node_features, W_read[:mem], W_read[mem:],
      b_read.reshape(1, mem), wmn)


def _edge_pre_body(ef_ref, te_ref, wme, wmt, bm, out_ref):
    out_ref[...] = (
        jnp.dot(ef_ref[...], wme[...], preferred_element_type=jnp.float32)
        + jnp.dot(te_ref[...], wmt[...], preferred_element_type=jnp.float32)
        + bm[...]
    )


def _edge_pre(edge_features, time_encoding, wme, wmt, b_msg, e_pad, blk):
    e, de = edge_features.shape
    dt = time_encoding.shape[1]
    mem = wme.shape[1]
    grid = e_pad // blk
    return pl.pallas_call(
        _edge_pre_body,
        grid=(grid,),
        in_specs=[
            pl.BlockSpec((blk, de), lambda i: (i, 0)),
            pl.BlockSpec((blk, dt), lambda i: (i, 0)),
            pl.BlockSpec((de, mem), lambda i: (0, 0)),
            pl.BlockSpec((dt, mem), lambda i: (0, 0)),
            pl.BlockSpec((1, mem), lambda i: (0, 0)),
        ],
        out_specs=pl.BlockSpec((blk, mem), lambda i: (i, 0)),
        out_shape=jax.ShapeDtypeStruct((e_pad, mem), jnp.float32),
    )(edge_features, time_encoding, wme, wmt, b_msg.reshape(1, mem))


def _make_sc_scatter(n_pad, mem, k_chunks, rows_per_tile):
    mesh = plsc.VectorSubcoreMesh(core_axis_name="c", subcore_axis_name="s")
    zero_full = rows_per_tile // CB
    zero_rem = rows_per_tile - zero_full * CB

    @functools.partial(
        pl.kernel,
        mesh=mesh,
        out_type=jax.ShapeDtypeStruct((2, n_pad, mem), jnp.float32),
        scratch_types=[
            pltpu.VMEM((CB,), jnp.int32),
            pltpu.VMEM((CB,), jnp.int32),
            pltpu.VMEM((CB, mem), jnp.float32),
            pltpu.VMEM((CB, mem), jnp.float32),
            pltpu.VMEM_SHARED((n_pad, mem), jnp.float32),
            pltpu.SemaphoreType.DMA,
        ],
    )
    def sc_scatter(npre_hbm, srcp_hbm, dstp_hbm, epre_hbm, accm_hbm,
                   src_v, dst_v, gbuf, ebuf, shm, sem):
        c = lax.axis_index("c")
        s = lax.axis_index("s")
        wid = c * 16 + s
        vregs = mem // LANES

        zvec = jnp.zeros((LANES,), jnp.float32)

        def zero_row(r, _):
            for k in range(vregs):
                ebuf[r, pl.ds(k * LANES, LANES)] = zvec
            return _

        lax.fori_loop(0, CB, zero_row, None)

        base_row = s * rows_per_tile
        for t in range(zero_full):
            pltpu.sync_copy(ebuf, shm.at[pl.ds(base_row + t * CB, CB)])
        if zero_rem:
            pltpu.sync_copy(ebuf.at[pl.ds(0, zero_rem)],
                            shm.at[pl.ds(base_row + zero_full * CB, zero_rem)])

        plsc.subcore_barrier()

        ebase = wid * (k_chunks * CB)

        def chunk_body(j, _):
            pltpu.sync_copy(srcp_hbm.at[wid * k_chunks + j], src_v)
            pltpu.sync_copy(dstp_hbm.at[wid * k_chunks + j], dst_v)
            gcopy = pltpu.async_copy(npre_hbm.at[src_v], gbuf, sem)
            pltpu.sync_copy(epre_hbm.at[pl.ds(ebase + j * CB, CB)], ebuf)
            gcopy.wait()

            def row_body(r, _):
                for k in range(vregs):
                    sl = pl.ds(k * LANES, LANES)
                    ebuf[r, sl] = jnp.maximum(ebuf[r, sl] + gbuf[r, sl], 0.0)
                return _

            lax.fori_loop(0, CB, row_body, None)

            pltpu.sync_copy(ebuf, shm.at[dst_v], add=True)
            return _

        lax.fori_loop(0, k_chunks, chunk_body, None)

        plsc.subcore_barrier()

        pltpu.sync_copy(shm.at[pl.ds(base_row, rows_per_tile)],
                        accm_hbm.at[c, pl.ds(base_row, rows_per_tile)])

    return sc_scatter


def _make_sc_count(n_pad, mem, k_chunks, rows_per_tile):
    mesh = plsc.VectorSubcoreMesh(core_axis_name="c", subcore_axis_name="s")
    zero_full = rows_per_tile // CB
    zero_rem = rows_per_tile - zero_full * CB

    @functools.partial(
        pl.kernel,
        mesh=mesh,
        out_type=jax.ShapeDtypeStruct((2, n_pad, mem), jnp.float32),
        scratch_types=[
            pltpu.VMEM((CB,), jnp.int32),
            pltpu.VMEM((CB, mem), jnp.float32),
            pltpu.VMEM_SHARED((n_pad, mem), jnp.float32),
        ],
    )
    def sc_count(dstp_hbm, accc_hbm, dst_v, obuf, shc):
        c = lax.axis_index("c")
        s = lax.axis_index("s")
        wid = c * 16 + s
        vregs = mem // LANES

        zvec = jnp.zeros((LANES,), jnp.float32)

        def zero_row(r, _):
            for k in range(vregs):
                obuf[r, pl.ds(k * LANES, LANES)] = zvec
            return _

        lax.fori_loop(0, CB, zero_row, None)

        base_row = s * rows_per_tile
        for t in range(zero_full):
            pltpu.sync_copy(obuf, shc.at[pl.ds(base_row + t * CB, CB)])
        if zero_rem:
            pltpu.sync_copy(obuf.at[pl.ds(0, zero_rem)],
                            shc.at[pl.ds(base_row + zero_full * CB, zero_rem)])

        onev = jnp.where(lax.iota(jnp.int32, LANES) == 0,
                         jnp.float32(1.0), jnp.float32(0.0))

        def ones_row(r, _):
            obuf[r, pl.ds(0, LANES)] = onev
            return _

        lax.fori_loop(0, CB, ones_row, None)

        plsc.subcore_barrier()

        def chunk_body(j, _):
            pltpu.sync_copy(dstp_hbm.at[wid * k_chunks + j], dst_v)
            pltpu.sync_copy(obuf, shc.at[dst_v], add=True)
            return _

        lax.fori_loop(0, k_chunks, chunk_body, None)

        plsc.subcore_barrier()

        pltpu.sync_copy(shc.at[pl.ds(base_row, rows_per_tile)],
                        accc_hbm.at[c, pl.ds(base_row, rows_per_tile)])

    return sc_count


def _final_body(accm_ref, accc_ref, nr_ref, mem_ref,
                wa1, wa2, ba, wu1, wu2, bu, wg1, wg2, bg, out_ref):
    cnt = accc_ref[0, :, 0:1] + accc_ref[1, :, 0:1]
    agg = (accm_ref[0] + accm_ref[1]) / jnp.maximum(cnt, 1.0)
    comb = jnp.maximum(
        jnp.dot(nr_ref[...], wa1[...], preferred_element_type=jnp.float32)
        + jnp.dot(agg, wa2[...], preferred_element_type=jnp.float32)
        + ba[...],
        0.0,
    )
    upd = jnp.tanh(
        jnp.dot(mem_ref[...], wu1[...], preferred_element_type=jnp.float32)
        + jnp.dot(comb, wu2[...], preferred_element_type=jnp.float32)
        + bu[...]
    )
    g_in = (
        jnp.dot(mem_ref[...], wg1[...], preferred_element_type=jnp.float32)
        + jnp.dot(upd, wg2[...], preferred_element_type=jnp.float32)
        + bg[...]
    )
    gate = 1.0 / (1.0 + jnp.exp(-g_in))
    out_ref[...] = gate * upd + (1.0 - gate) * mem_ref[...]


def _final(accm, accc, node_repr, node_memory,
           W_agg, b_agg, W_upd, b_upd, W_gate, b_gate, blk):
    n, mem = node_memory.shape
    grid = pl.cdiv(n, blk)
    wspec = pl.BlockSpec((mem, mem), lambda i: (0, 0))
    bspec = pl.BlockSpec((1, mem), lambda i: (0, 0))
    return pl.pallas_call(
        _final_body,
        grid=(grid,),
        in_specs=[
            pl.BlockSpec((2, blk, mem), lambda i: (0, i, 0)),
            pl.BlockSpec((2, blk, mem), lambda i: (0, i, 0)),
            pl.BlockSpec((blk, mem), lambda i: (i, 0)),
            pl.BlockSpec((blk, mem), lambda i: (i, 0)),
            wspec, wspec, bspec, wspec, wspec, bspec, wspec, wspec, bspec,
        ],
        out_specs=pl.BlockSpec((blk, mem), lambda i: (i, 0)),
        out_shape=jax.ShapeDtypeStruct((n, mem), jnp.float32),
    )(accm, accc, node_repr, node_memory,
      W_agg[:mem], W_agg[mem:], b_agg.reshape(1, mem),
      W_upd[:mem], W_upd[mem:], b_upd.reshape(1, mem),
      W_gate[:mem], W_gate[mem:], b_gate.reshape(1, mem))


def kernel(node_memory, node_features, edge_index, edge_features, time_encoding,
           W_read, b_read, W_msg, b_msg, W_agg, b_agg, W_upd, b_upd,
           W_gate, b_gate):
    n, mem = node_memory.shape
    e = edge_index.shape[1]
    de = edge_features.shape[1]

    ept = pl.cdiv(e, NW)
    k_chunks = pl.cdiv(ept, CB)
    ept = k_chunks * CB
    e_pad = NW * ept

    rows_per_tile = pl.cdiv(n + 1, 16 * 8) * 8
    n_pad = 16 * rows_per_tile

    src = edge_index[0]
    dst = edge_index[1]
    pad = e_pad - e
    srcp = jnp.concatenate(
        [src, jnp.zeros((pad,), jnp.int32)]).reshape(NW * k_chunks, CB)
    dstp = jnp.concatenate(
        [dst, jnp.full((pad,), n, jnp.int32)]).reshape(NW * k_chunks, CB)

    node_repr, node_pre = _node_prep(
        node_memory, node_features, W_read, b_read, W_msg[:mem], blk=1000)
    edge_pre = _edge_pre(
        edge_features, time_encoding,
        W_msg[mem:mem + de], W_msg[mem + de:], b_msg, e_pad, blk=4096)

    sc = _make_sc_scatter(n_pad, mem, k_chunks, rows_per_tile)
    accm = sc(node_pre, srcp, dstp, edge_pre)
    sc_cnt = _make_sc_count(n_pad, mem, k_chunks, rows_per_tile)
    accc = sc_cnt(dstp)

    return _final(accm, accc, node_repr, node_memory,
                  W_agg, b_agg, W_upd, b_upd, W_gate, b_gate, blk=1000)
